# Initial kernel scaffold; baseline (speedup 1.0000x reference)
import jax
import jax.numpy as jnp
from jax.experimental import pallas as pl


def kernel(x, neighbor_index, adjweight, Wc, bc, Wm, bm):
    raise NotImplementedError("write your pallas kernel here")


# speedup vs baseline: 16.5759x; 16.5759x over previous
import functools

import jax
import jax.numpy as jnp
from jax import lax
from jax.experimental import pallas as pl
from jax.experimental.pallas import tpu as pltpu
from jax.experimental.pallas import tpu_sc as plsc


def _elu(v):
    return jnp.where(v > 0, v, jnp.exp(jnp.minimum(v, 0.0)) - 1.0)


def _sc_gather(table, idx_flat):
    M = idx_flat.shape[0]
    F = table.shape[1]
    NW = 32
    m_per_w = M // NW
    CH = 400
    n_ch = m_per_w // CH
    mesh = plsc.VectorSubcoreMesh(core_axis_name="c", subcore_axis_name="s")

    @functools.partial(
        pl.kernel,
        mesh=mesh,
        out_type=jax.ShapeDtypeStruct((M, F), table.dtype),
        scratch_types=[
            pltpu.VMEM((CH,), jnp.int32),
            pltpu.VMEM((CH, F), table.dtype),
            pltpu.SemaphoreType.DMA,
        ],
    )
    def gather_kernel(table_hbm, idx_hbm, out_hbm, idx_v, rows_v, sem):
        wid = lax.axis_index("s") * 2 + lax.axis_index("c")
        base = wid * m_per_w

        @pl.loop(0, n_ch)
        def _(c):
            off = base + c * CH
            pltpu.sync_copy(idx_hbm.at[pl.ds(off, CH)], idx_v)
            pltpu.async_copy(table_hbm.at[idx_v], rows_v, sem).wait()
            pltpu.sync_copy(rows_v, out_hbm.at[pl.ds(off, CH)])

    return gather_kernel(table, idx_flat)


def _tc_compute(g, nbr2, adjw, x2, wcr, bc2, wmt, bm2, B, N, K, FIN, FOUT):
    NB = 400
    nblk = N // NB

    def body(g_ref, nbr_ref, adj_ref, x_ref, wcr_ref, bc_ref, wmt_ref, bm_ref,
             o_ref):
        i = pl.program_id(0)
        X = g_ref[...].reshape(NB, K, FIN)
        nbr = nbr_ref[...]
        Xm = X * (nbr != N - 1).astype(jnp.float32)[:, :, None]
        A = adj_ref[...]
        Y = lax.dot_general(A, Xm, (((1,), (1,)), ((0,), (0,))),
                            preferred_element_type=jnp.float32)
        acc = jnp.zeros((NB, FOUT), jnp.float32)
        for t in range(K):
            acc = acc + jnp.dot(_elu(Y[:, t, :]), wcr_ref[t],
                                preferred_element_type=jnp.float32)
        out_feat = _elu(acc + bc_ref[...])
        nidx = i * NB + lax.broadcasted_iota(jnp.int32, (NB, 1), 0)
        nmask = (nidx != N - 1).astype(jnp.float32)
        xm = x_ref[...] * nmask
        res = jnp.dot(xm, wmt_ref[...],
                      preferred_element_type=jnp.float32) + bm_ref[...]
        o_ref[...] = out_feat * nmask + res

    return pl.pallas_call(
        body,
        grid=(nblk, B),
        in_specs=[
            pl.BlockSpec((NB * K, FIN), lambda i, b: (b * nblk + i, 0)),
            pl.BlockSpec((NB, K), lambda i, b: (b * nblk + i, 0)),
            pl.BlockSpec((NB, K, K), lambda i, b: (i, 0, 0)),
            pl.BlockSpec((NB, FIN), lambda i, b: (b * nblk + i, 0)),
            pl.BlockSpec((K, FIN, FOUT), lambda i, b: (0, 0, 0)),
            pl.BlockSpec((1, FOUT), lambda i, b: (0, 0)),
            pl.BlockSpec((FIN, FOUT), lambda i, b: (0, 0)),
            pl.BlockSpec((1, FOUT), lambda i, b: (0, 0)),
        ],
        out_specs=pl.BlockSpec((NB, FOUT), lambda i, b: (b * nblk + i, 0)),
        out_shape=jax.ShapeDtypeStruct((B * N, FOUT), jnp.float32),
    )(g, nbr2, adjw, x2, wcr, bc2, wmt, bm2)


@jax.jit
def kernel(x, neighbor_index, adjweight, Wc, bc, Wm, bm):
    b, n, fin = x.shape
    k = neighbor_index.shape[-1]
    fout = Wc.shape[0]
    x2 = x.reshape(b * n, fin)
    nbr2 = neighbor_index.reshape(b * n, k).astype(jnp.int32)
    offs = (jnp.arange(b, dtype=jnp.int32) * n)[:, None, None]
    idx_flat = (neighbor_index.astype(jnp.int32) + offs).reshape(-1)
    g = _sc_gather(x2, idx_flat)
    wcr = Wc.reshape(fout, k, fin).transpose(1, 2, 0)
    wmt = Wm.T
    out2 = _tc_compute(g, nbr2, adjweight, x2, wcr, bc.reshape(1, -1), wmt,
                       bm.reshape(1, -1), b, n, k, fin, fout)
    return out2.reshape(b, n, fout)


# speedup vs baseline: 17.5014x; 1.0558x over previous
import functools

import jax
import jax.numpy as jnp
from jax import lax
from jax.experimental import pallas as pl
from jax.experimental.pallas import tpu as pltpu
from jax.experimental.pallas import tpu_sc as plsc


def _elu(v):
    return jnp.where(v > 0, v, jnp.exp(jnp.minimum(v, 0.0)) - 1.0)


def _sc_gather(table, idx_flat):
    M = idx_flat.shape[0]
    F = table.shape[1]
    NW = 32
    m_per_w = M // NW
    CH = 400
    n_ch = m_per_w // CH
    mesh = plsc.VectorSubcoreMesh(core_axis_name="c", subcore_axis_name="s")

    @functools.partial(
        pl.kernel,
        mesh=mesh,
        out_type=jax.ShapeDtypeStruct((M, F), table.dtype),
        scratch_types=[
            pltpu.VMEM((CH,), jnp.int32),
            pltpu.VMEM((CH, F), table.dtype),
            pltpu.SemaphoreType.DMA,
        ],
    )
    def gather_kernel(table_hbm, idx_hbm, out_hbm, idx_v, rows_v, sem):
        wid = lax.axis_index("s") * 2 + lax.axis_index("c")
        base = wid * m_per_w

        @pl.loop(0, n_ch)
        def _(c):
            off = base + c * CH
            pltpu.sync_copy(idx_hbm.at[pl.ds(off, CH)], idx_v)
            pltpu.async_copy(table_hbm.at[idx_v], rows_v, sem).wait()
            pltpu.sync_copy(rows_v, out_hbm.at[pl.ds(off, CH)])

    return gather_kernel(table, idx_flat)


def _tc_compute(g, nbr2, adjw, x2, wcr, bc2, wmt, bm2, B, N, K, FIN, FOUT):
    NB = 400
    nblk = N // NB

    def body(g_ref, nbr_ref, adj_ref, x_ref, wcr_ref, bc_ref, wmt_ref, bm_ref,
             o_ref):
        i = pl.program_id(0)
        X = g_ref[...].reshape(NB, K, FIN).astype(jnp.bfloat16)
        nbr = nbr_ref[...]
        A = adj_ref[...] * (nbr != N - 1).astype(jnp.bfloat16)[:, :, None]
        Y = lax.dot_general(A, X, (((1,), (1,)), ((0,), (0,))),
                            preferred_element_type=jnp.float32
                            ).astype(jnp.bfloat16)
        acc = jnp.zeros((NB, FOUT), jnp.float32)
        for t in range(K):
            acc = acc + jnp.dot(_elu(Y[:, t, :]), wcr_ref[t],
                                preferred_element_type=jnp.float32)
        out_feat = _elu(acc + bc_ref[...])
        nidx = i * NB + lax.broadcasted_iota(jnp.int32, (NB, 1), 0)
        nmask = (nidx != N - 1).astype(jnp.float32)
        xm = x_ref[...] * nmask
        res = jnp.dot(xm, wmt_ref[...],
                      preferred_element_type=jnp.float32) + bm_ref[...]
        o_ref[...] = out_feat * nmask + res

    return pl.pallas_call(
        body,
        grid=(nblk, B),
        in_specs=[
            pl.BlockSpec((NB * K, FIN), lambda i, b: (b * nblk + i, 0)),
            pl.BlockSpec((NB, K), lambda i, b: (b * nblk + i, 0)),
            pl.BlockSpec((NB, K, K), lambda i, b: (i, 0, 0)),
            pl.BlockSpec((NB, FIN), lambda i, b: (b * nblk + i, 0)),
            pl.BlockSpec((K, FIN, FOUT), lambda i, b: (0, 0, 0)),
            pl.BlockSpec((1, FOUT), lambda i, b: (0, 0)),
            pl.BlockSpec((FIN, FOUT), lambda i, b: (0, 0)),
            pl.BlockSpec((1, FOUT), lambda i, b: (0, 0)),
        ],
        out_specs=pl.BlockSpec((NB, FOUT), lambda i, b: (b * nblk + i, 0)),
        out_shape=jax.ShapeDtypeStruct((B * N, FOUT), jnp.float32),
        compiler_params=pltpu.CompilerParams(
            dimension_semantics=("parallel", "parallel")),
    )(g, nbr2, adjw, x2, wcr, bc2, wmt, bm2)


@jax.jit
def kernel(x, neighbor_index, adjweight, Wc, bc, Wm, bm):
    b, n, fin = x.shape
    k = neighbor_index.shape[-1]
    fout = Wc.shape[0]
    x2 = x.reshape(b * n, fin)
    nbr2 = neighbor_index.reshape(b * n, k).astype(jnp.int32)
    offs = (jnp.arange(b, dtype=jnp.int32) * n)[:, None, None]
    idx_flat = (neighbor_index.astype(jnp.int32) + offs).reshape(-1)
    g = _sc_gather(x2, idx_flat)
    wcr = Wc.reshape(fout, k, fin).transpose(1, 2, 0).astype(jnp.bfloat16)
    wmt = Wm.T
    out2 = _tc_compute(g, nbr2, adjweight.astype(jnp.bfloat16), x2, wcr,
                       bc.reshape(1, -1), wmt,
                       bm.reshape(1, -1), b, n, k, fin, fout)
    return out2.reshape(b, n, fout)


# speedup vs baseline: 18.5898x; 1.0622x over previous
import functools

import jax
import jax.numpy as jnp
from jax import lax
from jax.experimental import pallas as pl
from jax.experimental.pallas import tpu as pltpu
from jax.experimental.pallas import tpu_sc as plsc


def _elu(v):
    return jnp.where(v > 0, v, jnp.exp(jnp.minimum(v, 0.0)) - 1.0)


def _mask_pad_rows(x2, n_period):
    BN, F = x2.shape
    NBm = 2000

    def body(x_ref, o_ref):
        i = pl.program_id(0)
        r = i * NBm + lax.broadcasted_iota(jnp.int32, (NBm, 1), 0)
        keep = ((r + 1) % n_period != 0).astype(x_ref.dtype)
        o_ref[...] = x_ref[...] * keep

    return pl.pallas_call(
        body,
        grid=(BN // NBm,),
        in_specs=[pl.BlockSpec((NBm, F), lambda i: (i, 0))],
        out_specs=pl.BlockSpec((NBm, F), lambda i: (i, 0)),
        out_shape=jax.ShapeDtypeStruct((BN, F), x2.dtype),
    )(x2)


def _sc_gather(table, idx_flat):
    M = idx_flat.shape[0]
    F = table.shape[1]
    NW = 32
    m_per_w = M // NW
    CH = 400
    n_ch = m_per_w // CH
    mesh = plsc.VectorSubcoreMesh(core_axis_name="c", subcore_axis_name="s")

    @functools.partial(
        pl.kernel,
        mesh=mesh,
        out_type=jax.ShapeDtypeStruct((M, F), table.dtype),
        scratch_types=[
            pltpu.VMEM((CH,), jnp.int32),
            pltpu.VMEM((CH, F), table.dtype),
            pltpu.SemaphoreType.DMA,
        ],
    )
    def gather_kernel(table_hbm, idx_hbm, out_hbm, idx_v, rows_v, sem):
        wid = lax.axis_index("s") * 2 + lax.axis_index("c")
        base = wid * m_per_w

        @pl.loop(0, n_ch)
        def _(c):
            off = base + c * CH
            pltpu.sync_copy(idx_hbm.at[pl.ds(off, CH)], idx_v)
            pltpu.async_copy(table_hbm.at[idx_v], rows_v, sem).wait()
            pltpu.sync_copy(rows_v, out_hbm.at[pl.ds(off, CH)])

    return gather_kernel(table, idx_flat)


def _tc_compute(g, adjw, x2m, wcr, bc2, wmt, bm2, B, N, K, FIN, FOUT):
    NB = 400
    nblk = N // NB

    def body(g_ref, adj_ref, x_ref, wcr_ref, bc_ref, wmt_ref, bm_ref, o_ref):
        i = pl.program_id(0)
        X = g_ref[...].reshape(NB, K, FIN).astype(jnp.bfloat16)
        A = adj_ref[...]
        Y = lax.dot_general(A, X, (((1,), (1,)), ((0,), (0,))),
                            preferred_element_type=jnp.float32
                            ).astype(jnp.bfloat16)
        acc = jnp.zeros((NB, FOUT), jnp.float32)
        for t in range(K):
            acc = acc + jnp.dot(_elu(Y[:, t, :]), wcr_ref[t],
                                preferred_element_type=jnp.float32)
        out_feat = _elu(acc + bc_ref[...])
        nidx = i * NB + lax.broadcasted_iota(jnp.int32, (NB, 1), 0)
        nmask = (nidx != N - 1).astype(jnp.float32)
        res = jnp.dot(x_ref[...], wmt_ref[...],
                      preferred_element_type=jnp.float32) + bm_ref[...]
        o_ref[...] = out_feat * nmask + res

    return pl.pallas_call(
        body,
        grid=(nblk, B),
        in_specs=[
            pl.BlockSpec((NB * K, FIN), lambda i, b: (b * nblk + i, 0)),
            pl.BlockSpec((NB, K, K), lambda i, b: (i, 0, 0)),
            pl.BlockSpec((NB, FIN), lambda i, b: (b * nblk + i, 0)),
            pl.BlockSpec((K, FIN, FOUT), lambda i, b: (0, 0, 0)),
            pl.BlockSpec((1, FOUT), lambda i, b: (0, 0)),
            pl.BlockSpec((FIN, FOUT), lambda i, b: (0, 0)),
            pl.BlockSpec((1, FOUT), lambda i, b: (0, 0)),
        ],
        out_specs=pl.BlockSpec((NB, FOUT), lambda i, b: (b * nblk + i, 0)),
        out_shape=jax.ShapeDtypeStruct((B * N, FOUT), jnp.float32),
        compiler_params=pltpu.CompilerParams(
            dimension_semantics=("parallel", "parallel")),
    )(g, adjw, x2m, wcr, bc2, wmt, bm2)


@jax.jit
def kernel(x, neighbor_index, adjweight, Wc, bc, Wm, bm):
    b, n, fin = x.shape
    k = neighbor_index.shape[-1]
    fout = Wc.shape[0]
    x2 = x.reshape(b * n, fin)
    offs = (jnp.arange(b, dtype=jnp.int32) * n)[:, None, None]
    idx_flat = (neighbor_index.astype(jnp.int32) + offs).reshape(-1)
    x2m = _mask_pad_rows(x2, n)
    g = _sc_gather(x2m, idx_flat)
    wcr = Wc.reshape(fout, k, fin).transpose(1, 2, 0).astype(jnp.bfloat16)
    wmt = Wm.T
    out2 = _tc_compute(g, adjweight.astype(jnp.bfloat16), x2m, wcr,
                       bc.reshape(1, -1), wmt,
                       bm.reshape(1, -1), b, n, k, fin, fout)
    return out2.reshape(b, n, fout)
